# baseline (speedup 1.0000x reference)
import functools

import jax
import jax.numpy as jnp
from jax import lax
from jax.experimental import pallas as pl
from jax.experimental.pallas import tpu as pltpu
from jax.experimental.pallas import tpu_sc as plsc

POS = 200
BATCH = 1024
EMB = 64
FLAT = POS * BATCH

NC = 2
NS = 16
NW = NC * NS

SHI = POS // 8
BHI = BATCH // 128
UNITS = SHI * BHI
NBUF = 4

MAXU = 7


def _sc_body(tok_hbm, pe_hbm, table_hbm, out_hbm, idx_v, pe_v, *rest):
    bufs = rest[:NBUF]
    gsems = rest[NBUF:2 * NBUF]
    osems = rest[2 * NBUF:3 * NBUF]

    cid = lax.axis_index("c")
    sid = lax.axis_index("s")
    wid = cid * NS + sid

    ustart = jnp.where(wid < 8, 7 * wid, 56 + 6 * (wid - 8))
    ucnt = jnp.where(wid < 8, 7, 6)

    def load_tok(k, c):
        u = ustart + k
        pltpu.sync_copy(tok_hbm.at[u // BHI, u % BHI], idx_v.at[k])
        return c
    lax.fori_loop(0, ucnt, load_tok, 0)
    pltpu.sync_copy(pe_hbm, pe_v)

    def gather_descs(k, kblk, b):
        return [
            pltpu.make_async_copy(
                table_hbm.at[idx_v.at[k, 2 * kblk + h]],
                bufs[b].at[pl.ds(128 * h, 128)], gsems[b])
            for h in range(2)
        ]

    def out_descs(u, kblk, b):
        shi = u // BHI
        bhi = u % BHI
        descs = []
        for h in range(2):
            s = 8 * shi + 2 * kblk + h
            descs.append(pltpu.make_async_copy(
                bufs[b].at[pl.ds(128 * h, 128)],
                out_hbm.at[s, pl.ds(128 * bhi, 128)], osems[b]))
        return descs

    def unit(k):
        u = ustart + k
        for kblk in range(NBUF):
            @pl.when(k > 0)
            def _():
                for d in out_descs(u - 1, kblk, kblk):
                    d.wait()
            for d in gather_descs(k, kblk, kblk):
                d.start()
        for kblk in range(NBUF):
            for d in gather_descs(k, kblk, kblk):
                d.wait()
            for h in range(2):
                s = 8 * (u // BHI) + 2 * kblk + h

                def add_row(r, c2):
                    for cc in range(EMB // 16):
                        sl = pl.ds(cc * 16, 16)
                        plsc.addupdate(bufs[kblk].at[128 * h + r, sl],
                                       pe_v[s, sl])
                    return c2
                lax.fori_loop(0, 128, add_row, 0, unroll=8)
            for d in out_descs(u, kblk, kblk):
                d.start()

    def uloop(k, c):
        unit(k)
        return c
    lax.fori_loop(0, ucnt, uloop, 0)

    for kblk in range(NBUF):
        for d in out_descs(ustart + ucnt - 1, kblk, kblk):
            d.wait()


def _detile_body(tok_hbm, out_hbm, tile_v, sem):
    cid = lax.axis_index("c")
    sid = lax.axis_index("s")
    wid = cid * NS + sid
    ustart = jnp.where(wid < 8, 7 * wid, 56 + 6 * (wid - 8))
    ucnt = jnp.where(wid < 8, 7, 6)

    def unit(k, c):
        u = ustart + k
        shi = u // BHI
        bhi = u % BHI
        pltpu.sync_copy(
            tok_hbm.at[pl.ds(8 * shi, 8), pl.ds(128 * bhi, 128)], tile_v)
        pltpu.sync_copy(tile_v, out_hbm.at[shi, bhi])
        return c
    lax.fori_loop(0, ucnt, unit, 0)


@jax.jit
def _run(tok_t, pe2, table):
    detile = pl.kernel(
        _detile_body,
        out_type=jax.ShapeDtypeStruct((SHI, BHI, 8, 128), jnp.int32),
        mesh=plsc.VectorSubcoreMesh(core_axis_name="c", subcore_axis_name="s"),
        scratch_types=[
            pltpu.VMEM((8, 128), jnp.int32),
            pltpu.SemaphoreType.DMA,
        ],
    )
    tok4 = detile(tok_t)
    gather = pl.kernel(
        _sc_body,
        out_type=jax.ShapeDtypeStruct((POS, BATCH, EMB), jnp.float32),
        mesh=plsc.VectorSubcoreMesh(core_axis_name="c", subcore_axis_name="s"),
        scratch_types=(
            [pltpu.VMEM((MAXU, 8, 128), jnp.int32),
             pltpu.VMEM((POS, EMB), jnp.float32)]
            + [pltpu.VMEM((256, EMB), jnp.float32)] * NBUF
            + [pltpu.SemaphoreType.DMA] * (2 * NBUF)
        ),
        compiler_params=pltpu.CompilerParams(use_tc_tiling_on_sc=False),
    )
    return gather(tok4, pe2, table)


def kernel(tokens, embedding_table, positional_encoding):
    seq = tokens.shape[1]
    tok_t = tokens.T
    pe2 = positional_encoding[:seq]
    raw = _run(tok_t, pe2, embedding_table)
    return jnp.transpose(raw, (1, 0, 2))


# speedup vs baseline: 1.1076x; 1.1076x over previous
import functools

import jax
import jax.numpy as jnp
from jax import lax
from jax.experimental import pallas as pl
from jax.experimental.pallas import tpu as pltpu
from jax.experimental.pallas import tpu_sc as plsc

BATCH = 1024
SEQ = 200
EMB = 64

NC = 2
NS = 16
NW = NC * NS

SEQ_PER_W = BATCH // NW
GCHUNK = 100
CH_PER_SEQ = SEQ // GCHUNK
CH_PER_W = SEQ_PER_W * CH_PER_SEQ
NBUF = 4
LEAD = 2


def _tok_encode_body(tokens_hbm, pe_hbm, table_hbm, out_hbm,
                     idx_v, pe_v, *rest):
    bufs = rest[:NBUF]
    gsems = rest[NBUF:2 * NBUF]
    osems = rest[2 * NBUF:3 * NBUF]

    cid = lax.axis_index("c")
    sid = lax.axis_index("s")
    wid = cid * NS + sid

    pltpu.sync_copy(tokens_hbm.at[wid], idx_v)
    pltpu.sync_copy(pe_hbm, pe_v)

    def gather_descs(seq, b):
        ch = seq * CH_PER_SEQ
        return [
            pltpu.make_async_copy(
                table_hbm.at[idx_v.at[ch]],
                bufs[b].at[pl.ds(0, GCHUNK)], gsems[b]),
            pltpu.make_async_copy(
                table_hbm.at[idx_v.at[ch + 1]],
                bufs[b].at[pl.ds(GCHUNK, GCHUNK)], gsems[b]),
        ]

    def out_desc(seq, b):
        return pltpu.make_async_copy(
            bufs[b], out_hbm.at[wid * SEQ_PER_W + seq], osems[b])

    for t in range(LEAD):
        for d in gather_descs(t, t % NBUF):
            d.start()

    def turn(t, b):
        nxt = t + LEAD

        @pl.when(nxt < SEQ_PER_W)
        def _():
            bb = (b + LEAD) % NBUF

            @pl.when(nxt >= NBUF)
            def _():
                out_desc(nxt - NBUF, bb).wait()
            for d in gather_descs(nxt, bb):
                d.start()

        for d in gather_descs(t, b):
            d.wait()

        def add_row(r, c2):
            for cc in range(EMB // 16):
                sl = pl.ds(cc * 16, 16)
                plsc.addupdate(bufs[b].at[r, sl], pe_v[r, sl])
            return c2
        lax.fori_loop(0, SEQ, add_row, 0, unroll=8)

        out_desc(t, b).start()

    def outer(g, carry):
        for b in range(NBUF):
            turn(g * NBUF + b, b)
        return carry

    lax.fori_loop(0, SEQ_PER_W // NBUF, outer, 0)

    for t in range(SEQ_PER_W - NBUF, SEQ_PER_W):
        out_desc(t, t % NBUF).wait()


@jax.jit
def _run(tokens_r, pe, table):
    f = pl.kernel(
        _tok_encode_body,
        out_type=jax.ShapeDtypeStruct((BATCH, SEQ, EMB), jnp.float32),
        mesh=plsc.VectorSubcoreMesh(core_axis_name="c", subcore_axis_name="s"),
        scratch_types=(
            [pltpu.VMEM((CH_PER_W, GCHUNK), jnp.int32),
             pltpu.VMEM((SEQ, EMB), jnp.float32)]
            + [pltpu.VMEM((SEQ, EMB), jnp.float32)] * NBUF
            + [pltpu.SemaphoreType.DMA] * (2 * NBUF)
        ),
        compiler_params=pltpu.CompilerParams(use_tc_tiling_on_sc=False),
    )
    return f(tokens_r, pe, table)


def kernel(tokens, embedding_table, positional_encoding):
    seq = tokens.shape[1]
    tokens_r = tokens.reshape(NW, CH_PER_W, GCHUNK)
    pe = positional_encoding[:seq]
    return _run(tokens_r, pe, embedding_table)


# speedup vs baseline: 1.1099x; 1.0020x over previous
import functools

import jax
import jax.numpy as jnp
from jax import lax
from jax.experimental import pallas as pl
from jax.experimental.pallas import tpu as pltpu
from jax.experimental.pallas import tpu_sc as plsc

BATCH = 1024
SEQ = 200
EMB = 64

NC = 2
NS = 16
NW = NC * NS

SEQ_PER_W = BATCH // NW
GCHUNK = 100
CH_PER_SEQ = SEQ // GCHUNK
CH_PER_W = SEQ_PER_W * CH_PER_SEQ
NBUF = 4
LEAD = 2


def _tok_encode_body(tokens_hbm, pe_hbm, table_hbm, out_hbm,
                     idx_v, pe_v, *rest):
    bufs = rest[:NBUF]
    gsems = rest[NBUF:2 * NBUF]
    osems = rest[2 * NBUF:3 * NBUF]

    cid = lax.axis_index("c")
    sid = lax.axis_index("s")
    wid = cid * NS + sid

    pltpu.sync_copy(tokens_hbm.at[wid], idx_v)
    pltpu.sync_copy(pe_hbm, pe_v)

    def gather_descs(seq, b):
        ch = seq * CH_PER_SEQ
        return [
            pltpu.make_async_copy(
                table_hbm.at[idx_v.at[ch]],
                bufs[b].at[pl.ds(0, GCHUNK)], gsems[b]),
            pltpu.make_async_copy(
                table_hbm.at[idx_v.at[ch + 1]],
                bufs[b].at[pl.ds(GCHUNK, GCHUNK)], gsems[b]),
        ]

    def out_desc(seq, b):
        return pltpu.make_async_copy(
            bufs[b], out_hbm.at[pl.ds((wid * SEQ_PER_W + seq) * SEQ, SEQ)],
            osems[b])

    for t in range(LEAD):
        for d in gather_descs(t, t % NBUF):
            d.start()

    def turn(t, b):
        nxt = t + LEAD

        @pl.when(nxt < SEQ_PER_W)
        def _():
            bb = (b + LEAD) % NBUF

            @pl.when(nxt >= NBUF)
            def _():
                out_desc(nxt - NBUF, bb).wait()
            for d in gather_descs(nxt, bb):
                d.start()

        for d in gather_descs(t, b):
            d.wait()

        def add_row(r, c2):
            for cc in range(EMB // 16):
                sl = pl.ds(cc * 16, 16)
                plsc.addupdate(bufs[b].at[r, sl], pe_v[r, sl])
            return c2
        lax.fori_loop(0, SEQ, add_row, 0, unroll=8)

        out_desc(t, b).start()

    def outer(g, carry):
        for b in range(NBUF):
            turn(g * NBUF + b, b)
        return carry

    lax.fori_loop(0, SEQ_PER_W // NBUF, outer, 0)

    for t in range(SEQ_PER_W - NBUF, SEQ_PER_W):
        out_desc(t, t % NBUF).wait()


@jax.jit
def _run(tokens_r, pe, table):
    f = pl.kernel(
        _tok_encode_body,
        out_type=jax.ShapeDtypeStruct((BATCH * SEQ, EMB), jnp.float32),
        mesh=plsc.VectorSubcoreMesh(core_axis_name="c", subcore_axis_name="s"),
        scratch_types=(
            [pltpu.VMEM((CH_PER_W, GCHUNK), jnp.int32),
             pltpu.VMEM((SEQ, EMB), jnp.float32)]
            + [pltpu.VMEM((SEQ, EMB), jnp.float32)] * NBUF
            + [pltpu.SemaphoreType.DMA] * (2 * NBUF)
        ),
        compiler_params=pltpu.CompilerParams(use_tc_tiling_on_sc=False),
    )
    return f(tokens_r, pe, table)


def kernel(tokens, embedding_table, positional_encoding):
    seq = tokens.shape[1]
    tokens_r = tokens.reshape(NW, CH_PER_W, GCHUNK)
    pe = positional_encoding[:seq]
    return _run(tokens_r, pe, embedding_table).reshape(BATCH, SEQ, EMB)
